# baseline (speedup 1.0000x reference)
import functools

import jax
import jax.numpy as jnp
from jax import lax
from jax.experimental import pallas as pl
from jax.experimental.pallas import tpu as pltpu
from jax.experimental.pallas import tpu_sc as plsc

_N = 10000
_M = 20000
_P = 320000
_D = 128
_H = 64
_LANES = 128
_NS = 16
_NC = 2
_RPT = 160
_IDX_ROWS = _RPT * _NS
_P_PAD = _IDX_ROWS * _LANES
_MG = 20224
_NG = 10112
_MZ = _MG // _NS
_NZ = _NG // _NS
_CROWS = 40
_NCH = _RPT // _CROWS
_CW = 8


def _sc_mesh():
    return plsc.VectorSubcoreMesh(
        core_axis_name="c", subcore_axis_name="s",
        num_cores=_NC, num_subcores=_NS)


def _stream_pairs(src, gidx, sidx, acc_s, rows, sems, extra=None):
    pltpu.async_copy(src.at[gidx.at[0]], rows[0], sems[0])
    pltpu.async_copy(src.at[gidx.at[1]], rows[1], sems[1])

    def body(jj, carry):
        for b in range(3):
            s = jj * 3 + b
            nxt = jnp.minimum(s + 2, _CROWS - 1)
            pltpu.async_copy(
                src.at[gidx.at[nxt]], rows[(b + 2) % 3], sems[(b + 2) % 3])
            pltpu.make_async_copy(
                src.at[pl.ds(0, _LANES)], rows[b], sems[b]).wait()
            pltpu.sync_copy(rows[b], acc_s.at[sidx.at[s]], add=True)
            if extra is not None:
                extra(s)
        return carry

    lax.fori_loop(0, (_CROWS - 1) // 3, body, 0)
    pltpu.make_async_copy(src.at[pl.ds(0, _LANES)], rows[0], sems[0]).wait()
    pltpu.sync_copy(rows[0], acc_s.at[sidx.at[_CROWS - 1]], add=True)
    if extra is not None:
        extra(_CROWS - 1)
    pltpu.make_async_copy(src.at[pl.ds(0, _LANES)], rows[1], sems[1]).wait()


@functools.partial(
    pl.kernel,
    out_type=(jax.ShapeDtypeStruct((2 * _MG, _H), jnp.float32),
              jax.ShapeDtypeStruct((2 * _NG, _H), jnp.float32)),
    mesh=_sc_mesh(),
    compiler_params=pltpu.CompilerParams(
        use_tc_tiling_on_sc=False, needs_layout_passes=False),
    scratch_types=[
        pltpu.VMEM_SHARED((_MG, _H), jnp.float32),
        pltpu.VMEM_SHARED((_MG, _CW), jnp.float32),
        pltpu.VMEM((_CROWS, _LANES), jnp.int32),
        pltpu.VMEM((_CROWS, _LANES), jnp.int32),
        pltpu.VMEM((_LANES, _H), jnp.float32),
        pltpu.VMEM((_LANES, _H), jnp.float32),
        pltpu.VMEM((_LANES, _H), jnp.float32),
        pltpu.VMEM((_LANES, _CW), jnp.float32),
        pltpu.SemaphoreType.DMA,
        pltpu.SemaphoreType.DMA,
        pltpu.SemaphoreType.DMA,
    ],
)
def _kmain(xcat, vtx_g, edg_s, edg_g, vtx_s, ones_h, z64, zc,
           xe_out, xv_out,
           acc_s, counts_s, idx_a, idx_b, rows0, rows1, rows2, aux,
           sem0, sem1, sem2):
    cid = lax.axis_index("c")
    sid = lax.axis_index("s")
    rows = (rows0, rows1, rows2)
    sems = (sem0, sem1, sem2)

    pltpu.sync_copy(z64, acc_s.at[pl.ds(sid * _MZ, _MZ)])
    pltpu.sync_copy(zc, counts_s.at[pl.ds(sid * _MZ, _MZ)])
    pltpu.sync_copy(ones_h, aux)
    plsc.subcore_barrier()

    for ci in range(_NCH):
        base = sid * _RPT + ci * _CROWS
        pltpu.sync_copy(vtx_g.at[pl.ds(cid * _IDX_ROWS + base, _CROWS)], idx_a)
        pltpu.sync_copy(edg_s.at[pl.ds(base, _CROWS)], idx_b)

        def extra(s):
            pltpu.sync_copy(aux, counts_s.at[idx_b.at[s]], add=True)

        _stream_pairs(xcat, idx_a, idx_b, acc_s, rows, sems, extra)

    plsc.subcore_barrier()

    iota16 = lax.iota(jnp.int32, 16)
    zero16 = jnp.zeros((16,), jnp.int32)
    for k in range(10):
        rcnt = 128 if k < 9 else _MZ - 9 * 128
        off = sid * _MZ + k * 128
        pltpu.sync_copy(acc_s.at[pl.ds(off, rcnt)], rows0.at[pl.ds(0, rcnt)])
        pltpu.sync_copy(counts_s.at[pl.ds(off, rcnt)], aux.at[pl.ds(0, rcnt)])

        def dbody(g, carry):
            ridx = g * 16 + iota16
            cnt = plsc.load_gather(aux, [ridx, zero16])
            rcp = 1.0 / jnp.maximum(cnt, 1.0)
            for c in range(_H):
                cidx = jnp.full((16,), c, jnp.int32)
                v = plsc.load_gather(rows0, [ridx, cidx])
                plsc.store_scatter(rows0, [ridx, cidx], v * rcp)
            return carry

        lax.fori_loop(0, rcnt // 16, dbody, 0)
        pltpu.sync_copy(rows0.at[pl.ds(0, rcnt)],
                        xe_out.at[pl.ds(cid * _MG + off, rcnt)])

    plsc.subcore_barrier()

    pltpu.sync_copy(z64.at[pl.ds(0, _NZ)], acc_s.at[pl.ds(sid * _NZ, _NZ)])
    plsc.subcore_barrier()

    for ci in range(_NCH):
        base = sid * _RPT + ci * _CROWS
        pltpu.sync_copy(edg_g.at[pl.ds(cid * _IDX_ROWS + base, _CROWS)], idx_a)
        pltpu.sync_copy(vtx_s.at[pl.ds(base, _CROWS)], idx_b)
        _stream_pairs(xe_out, idx_a, idx_b, acc_s, rows, sems)

    plsc.subcore_barrier()
    pltpu.sync_copy(
        acc_s.at[pl.ds(sid * _NZ, _NZ)],
        xv_out.at[pl.ds(cid * _NG + sid * _NZ, _NZ)])


_BN4 = 1000


def _k4_body(x_ref, lo_ref, hi_ref, w_ref, e_ref, o_ref):
    xn = e_ref[0, 0] * x_ref[:, :] + jnp.concatenate(
        [lo_ref[:, :], hi_ref[:, :]], axis=1)
    o_ref[:, :] = lax.dot_general(
        xn, w_ref[:, :], (((1,), (1,)), ((), ())),
        preferred_element_type=jnp.float32)


def _k4(X, xv_lo, xv_hi, W, epsp1):
    nb = _N // _BN4
    return pl.pallas_call(
        _k4_body,
        grid=(nb,),
        in_specs=[
            pl.BlockSpec((_BN4, _D), lambda i: (i, 0)),
            pl.BlockSpec((_BN4, _H), lambda i: (i, 0)),
            pl.BlockSpec((_BN4, _H), lambda i: (i, 0)),
            pl.BlockSpec((_D, _D), lambda i: (0, 0)),
            pl.BlockSpec(memory_space=pltpu.SMEM),
        ],
        out_specs=pl.BlockSpec((_BN4, _D), lambda i: (i, 0)),
        out_shape=jax.ShapeDtypeStruct((_N, _D), jnp.float32),
    )(X, xv_lo, xv_hi, W, epsp1)


def kernel(X, vertex, edges, W, eps):
    pad = _P_PAD - _P
    ar = jnp.arange(pad, dtype=jnp.int32)
    vtx_g = jnp.concatenate([vertex, (ar * 7919) % _N])
    vtx_g2 = jnp.concatenate([vtx_g, vtx_g + _N]).reshape(2 * _IDX_ROWS, _LANES)
    edg_s = jnp.concatenate([edges, _M + (ar % (_MG - _M))]).reshape(
        _IDX_ROWS, _LANES)
    edg_g = jnp.concatenate([edges, (ar * 7919) % _M])
    edg_g2 = jnp.concatenate([edg_g, edg_g + _MG]).reshape(
        2 * _IDX_ROWS, _LANES)
    vtx_s = jnp.concatenate([vertex, _N + (ar % (_NG - _N))]).reshape(
        _IDX_ROWS, _LANES)

    xcat = jnp.concatenate([X[:, :_H], X[:, _H:]], axis=0)
    ones_h = jnp.ones((_LANES, _CW), jnp.float32)
    z64 = jnp.zeros((_MZ, _H), jnp.float32)
    zc = jnp.zeros((_MZ, _CW), jnp.float32)

    _, xv_cat = _kmain(xcat, vtx_g2, edg_s, edg_g2, vtx_s, ones_h, z64, zc)
    xv_lo = xv_cat[:_N]
    xv_hi = xv_cat[_NG:_NG + _N]
    epsp1 = (1.0 + eps).reshape(1, 1)
    return _k4(X, xv_lo, xv_hi, W, epsp1)


# speedup vs baseline: 1.2540x; 1.2540x over previous
import functools

import jax
import jax.numpy as jnp
from jax import lax
from jax.experimental import pallas as pl
from jax.experimental.pallas import tpu as pltpu
from jax.experimental.pallas import tpu_sc as plsc

_N = 10000
_M = 20000
_P = 320000
_D = 128
_H = 64
_LANES = 128
_NS = 16
_NC = 2
_RPT = 160
_IDX_ROWS = _RPT * _NS
_P_PAD = _IDX_ROWS * _LANES
_MG = 20224
_NG = 10112
_MZ = _MG // _NS
_NZ = _NG // _NS
_CROWS = 40
_NCH = _RPT // _CROWS
_CW = 8


def _sc_mesh():
    return plsc.VectorSubcoreMesh(
        core_axis_name="c", subcore_axis_name="s",
        num_cores=_NC, num_subcores=_NS)


def _stream_pairs(src, gidx, sidx, acc_s, rows, sems, extra=None):
    pltpu.async_copy(src.at[gidx.at[0]], rows[0], sems[0])
    pltpu.async_copy(src.at[gidx.at[1]], rows[1], sems[1])

    def body(jj, carry):
        for b in range(3):
            s = jj * 3 + b
            nxt = jnp.minimum(s + 2, _CROWS - 1)
            pltpu.async_copy(
                src.at[gidx.at[nxt]], rows[(b + 2) % 3], sems[(b + 2) % 3])
            pltpu.make_async_copy(
                src.at[pl.ds(0, _LANES)], rows[b], sems[b]).wait()
            pltpu.sync_copy(rows[b], acc_s.at[sidx.at[s]], add=True)
            if extra is not None:
                extra(s)
        return carry

    lax.fori_loop(0, (_CROWS - 1) // 3, body, 0)
    pltpu.make_async_copy(src.at[pl.ds(0, _LANES)], rows[0], sems[0]).wait()
    pltpu.sync_copy(rows[0], acc_s.at[sidx.at[_CROWS - 1]], add=True)
    if extra is not None:
        extra(_CROWS - 1)
    pltpu.make_async_copy(src.at[pl.ds(0, _LANES)], rows[1], sems[1]).wait()


@functools.partial(
    pl.kernel,
    out_type=(jax.ShapeDtypeStruct((2 * _MG, _H), jnp.float32),
              jax.ShapeDtypeStruct((2 * _NG, _H), jnp.float32)),
    mesh=_sc_mesh(),
    compiler_params=pltpu.CompilerParams(
        use_tc_tiling_on_sc=False, needs_layout_passes=False),
    scratch_types=[
        pltpu.VMEM_SHARED((_MG, _H), jnp.float32),
        pltpu.VMEM_SHARED((_MG, _CW), jnp.float32),
        pltpu.VMEM((_CROWS, _LANES), jnp.int32),
        pltpu.VMEM((_CROWS, _LANES), jnp.int32),
        pltpu.VMEM((_LANES, _H), jnp.float32),
        pltpu.VMEM((_LANES, _H), jnp.float32),
        pltpu.VMEM((_LANES, _H), jnp.float32),
        pltpu.VMEM((_LANES, _CW), jnp.float32),
        pltpu.SemaphoreType.DMA,
        pltpu.SemaphoreType.DMA,
        pltpu.SemaphoreType.DMA,
        pltpu.SemaphoreType.DMA,
        pltpu.SemaphoreType.DMA,
    ],
)
def _kmain(xcat, vtx_g, edg_s, edg_g, vtx_s, ones_h, z64, zc,
           xe_out, xv_out,
           acc_s, counts_s, idx_a, idx_b, rows0, rows1, rows2, aux,
           sem0, sem1, sem2, wsem0, wsem1):
    cid = lax.axis_index("c")
    sid = lax.axis_index("s")
    rows = (rows0, rows1, rows2)
    sems = (sem0, sem1, sem2)

    pltpu.sync_copy(z64, acc_s.at[pl.ds(sid * _MZ, _MZ)])
    pltpu.sync_copy(zc, counts_s.at[pl.ds(sid * _MZ, _MZ)])
    pltpu.sync_copy(ones_h, aux)
    plsc.subcore_barrier()

    for ci in range(_NCH):
        base = sid * _RPT + ci * _CROWS
        pltpu.sync_copy(vtx_g.at[pl.ds(cid * _IDX_ROWS + base, _CROWS)], idx_a)
        pltpu.sync_copy(edg_s.at[pl.ds(base, _CROWS)], idx_b)

        def extra(s):
            pltpu.sync_copy(aux, counts_s.at[idx_b.at[s]], add=True)

        _stream_pairs(xcat, idx_a, idx_b, acc_s, rows, sems, extra)

    plsc.subcore_barrier()

    iota16 = lax.iota(jnp.int32, 16)
    zero16 = jnp.zeros((16,), jnp.int32)
    dbufs = (rows0, rows1)
    wsems = (wsem0, wsem1)
    for k in range(10):
        b = k % 2
        rcnt = 128 if k < 9 else _MZ - 9 * 128
        off = sid * _MZ + k * 128
        if k >= 2:
            pltpu.make_async_copy(
                xe_out.at[pl.ds(0, 128)], dbufs[b].at[pl.ds(0, 128)],
                wsems[b]).wait()
        pltpu.sync_copy(acc_s.at[pl.ds(off, rcnt)], dbufs[b].at[pl.ds(0, rcnt)])
        pltpu.sync_copy(counts_s.at[pl.ds(off, rcnt)], aux.at[pl.ds(0, rcnt)])

        def dbody(g, carry, _b=b):
            cnt = plsc.load_gather(aux, [g * 16 + iota16, zero16])
            rcp = 1.0 / jnp.maximum(cnt, 1.0)
            for i in range(16):
                r = g * 16 + i
                rc = rcp[i]
                for c in range(_H // 16):
                    sl = pl.ds(c * 16, 16)
                    dbufs[_b][r, sl] = dbufs[_b][r, sl] * rc
            return carry

        lax.fori_loop(0, rcnt // 16, dbody, 0)
        pltpu.async_copy(dbufs[b].at[pl.ds(0, rcnt)],
                         xe_out.at[pl.ds(cid * _MG + off, rcnt)], wsems[b])
    pltpu.make_async_copy(
        xe_out.at[pl.ds(0, 128)], rows0.at[pl.ds(0, 128)], wsem0).wait()
    pltpu.make_async_copy(
        xe_out.at[pl.ds(0, _MZ - 9 * 128)],
        rows1.at[pl.ds(0, _MZ - 9 * 128)], wsem1).wait()

    plsc.subcore_barrier()

    pltpu.sync_copy(z64.at[pl.ds(0, _NZ)], acc_s.at[pl.ds(sid * _NZ, _NZ)])
    plsc.subcore_barrier()

    for ci in range(_NCH):
        base = sid * _RPT + ci * _CROWS
        pltpu.sync_copy(edg_g.at[pl.ds(cid * _IDX_ROWS + base, _CROWS)], idx_a)
        pltpu.sync_copy(vtx_s.at[pl.ds(base, _CROWS)], idx_b)
        _stream_pairs(xe_out, idx_a, idx_b, acc_s, rows, sems)

    plsc.subcore_barrier()
    pltpu.sync_copy(
        acc_s.at[pl.ds(sid * _NZ, _NZ)],
        xv_out.at[pl.ds(cid * _NG + sid * _NZ, _NZ)])


_BN4 = 1000


def _k4_body(x_ref, lo_ref, hi_ref, w_ref, e_ref, o_ref):
    xn = e_ref[0, 0] * x_ref[:, :] + jnp.concatenate(
        [lo_ref[:, :], hi_ref[:, :]], axis=1)
    o_ref[:, :] = lax.dot_general(
        xn, w_ref[:, :], (((1,), (1,)), ((), ())),
        preferred_element_type=jnp.float32)


def _k4(X, xv_lo, xv_hi, W, epsp1):
    nb = _N // _BN4
    return pl.pallas_call(
        _k4_body,
        grid=(nb,),
        in_specs=[
            pl.BlockSpec((_BN4, _D), lambda i: (i, 0)),
            pl.BlockSpec((_BN4, _H), lambda i: (i, 0)),
            pl.BlockSpec((_BN4, _H), lambda i: (i, 0)),
            pl.BlockSpec((_D, _D), lambda i: (0, 0)),
            pl.BlockSpec(memory_space=pltpu.SMEM),
        ],
        out_specs=pl.BlockSpec((_BN4, _D), lambda i: (i, 0)),
        out_shape=jax.ShapeDtypeStruct((_N, _D), jnp.float32),
    )(X, xv_lo, xv_hi, W, epsp1)


def kernel(X, vertex, edges, W, eps):
    pad = _P_PAD - _P
    ar = jnp.arange(pad, dtype=jnp.int32)
    vtx_g = jnp.concatenate([vertex, (ar * 7919) % _N])
    vtx_g2 = jnp.concatenate([vtx_g, vtx_g + _N]).reshape(2 * _IDX_ROWS, _LANES)
    edg_s = jnp.concatenate([edges, _M + (ar % (_MG - _M))]).reshape(
        _IDX_ROWS, _LANES)
    edg_g = jnp.concatenate([edges, (ar * 7919) % _M])
    edg_g2 = jnp.concatenate([edg_g, edg_g + _MG]).reshape(
        2 * _IDX_ROWS, _LANES)
    vtx_s = jnp.concatenate([vertex, _N + (ar % (_NG - _N))]).reshape(
        _IDX_ROWS, _LANES)

    xcat = jnp.concatenate([X[:, :_H], X[:, _H:]], axis=0)
    ones_h = jnp.ones((_LANES, _CW), jnp.float32)
    z64 = jnp.zeros((_MZ, _H), jnp.float32)
    zc = jnp.zeros((_MZ, _CW), jnp.float32)

    _, xv_cat = _kmain(xcat, vtx_g2, edg_s, edg_g2, vtx_s, ones_h, z64, zc)
    xv_lo = xv_cat[:_N]
    xv_hi = xv_cat[_NG:_NG + _N]
    epsp1 = (1.0 + eps).reshape(1, 1)
    return _k4(X, xv_lo, xv_hi, W, epsp1)


# speedup vs baseline: 1.2812x; 1.0217x over previous
import functools

import jax
import jax.numpy as jnp
from jax import lax
from jax.experimental import pallas as pl
from jax.experimental.pallas import tpu as pltpu
from jax.experimental.pallas import tpu_sc as plsc

_N = 10000
_M = 20000
_P = 320000
_D = 128
_H = 64
_LANES = 128
_NS = 16
_NC = 2
_RPT = 160
_IDX_ROWS = _RPT * _NS
_P_PAD = _IDX_ROWS * _LANES
_MG = 20224
_NG = 10112
_MZ = _MG // _NS
_NZ = _NG // _NS
_CROWS = 40
_NCH = _RPT // _CROWS
_CW = 8


def _sc_mesh():
    return plsc.VectorSubcoreMesh(
        core_axis_name="c", subcore_axis_name="s",
        num_cores=_NC, num_subcores=_NS)


def _stream_pairs(src, gidx, sidx, acc_s, rows, sems, extra=None):
    pltpu.async_copy(src.at[gidx.at[0]], rows[0], sems[0])
    pltpu.async_copy(src.at[gidx.at[1]], rows[1], sems[1])

    def body(jj, carry):
        for b in range(3):
            s = jj * 3 + b
            nxt = jnp.minimum(s + 2, _CROWS - 1)
            pltpu.async_copy(
                src.at[gidx.at[nxt]], rows[(b + 2) % 3], sems[(b + 2) % 3])
            pltpu.make_async_copy(
                src.at[pl.ds(0, _LANES)], rows[b], sems[b]).wait()
            pltpu.sync_copy(rows[b], acc_s.at[sidx.at[s]], add=True)
            if extra is not None:
                extra(s)
        return carry

    lax.fori_loop(0, (_CROWS - 1) // 3, body, 0)
    pltpu.make_async_copy(src.at[pl.ds(0, _LANES)], rows[0], sems[0]).wait()
    pltpu.sync_copy(rows[0], acc_s.at[sidx.at[_CROWS - 1]], add=True)
    if extra is not None:
        extra(_CROWS - 1)
    pltpu.make_async_copy(src.at[pl.ds(0, _LANES)], rows[1], sems[1]).wait()


@functools.partial(
    pl.kernel,
    out_type=(jax.ShapeDtypeStruct((2 * _MG, _H), jnp.float32),
              jax.ShapeDtypeStruct((2 * _NG, _H), jnp.float32)),
    mesh=_sc_mesh(),
    compiler_params=pltpu.CompilerParams(
        use_tc_tiling_on_sc=False, needs_layout_passes=False),
    scratch_types=[
        pltpu.VMEM_SHARED((_MG, _H), jnp.float32),
        pltpu.VMEM_SHARED((_MG, _CW), jnp.float32),
        pltpu.VMEM((_CROWS, _LANES), jnp.int32),
        pltpu.VMEM((_CROWS, _LANES), jnp.int32),
        pltpu.VMEM((_LANES, _H), jnp.float32),
        pltpu.VMEM((_LANES, _H), jnp.float32),
        pltpu.VMEM((_LANES, _H), jnp.float32),
        pltpu.VMEM((_LANES, _CW), jnp.float32),
        pltpu.SemaphoreType.DMA,
        pltpu.SemaphoreType.DMA,
        pltpu.SemaphoreType.DMA,
        pltpu.SemaphoreType.DMA,
        pltpu.SemaphoreType.DMA,
    ],
)
def _kmain(xcat, vtx_g, edg_s, edg_g, vtx_s, ones_h, z64, zc,
           xe_out, xv_out,
           acc_s, counts_s, idx_a, idx_b, rows0, rows1, rows2, aux,
           sem0, sem1, sem2, wsem0, wsem1):
    cid = lax.axis_index("c")
    sid = lax.axis_index("s")
    rows = (rows0, rows1, rows2)
    sems = (sem0, sem1, sem2)

    pltpu.sync_copy(z64, rows0)
    pltpu.sync_copy(zc, aux)
    for k in range(10):
        rcnt = 128 if k < 9 else _MZ - 9 * 128
        off = sid * _MZ + k * 128
        pltpu.sync_copy(rows0.at[pl.ds(0, rcnt)], acc_s.at[pl.ds(off, rcnt)])
        pltpu.sync_copy(aux.at[pl.ds(0, rcnt)], counts_s.at[pl.ds(off, rcnt)])
    pltpu.sync_copy(ones_h, aux)
    plsc.subcore_barrier()

    for ci in range(_NCH):
        base = sid * _RPT + ci * _CROWS
        pltpu.sync_copy(vtx_g.at[pl.ds(cid * _IDX_ROWS + base, _CROWS)], idx_a)
        pltpu.sync_copy(edg_s.at[pl.ds(base, _CROWS)], idx_b)

        def extra(s):
            pltpu.sync_copy(aux, counts_s.at[idx_b.at[s]], add=True)

        _stream_pairs(xcat, idx_a, idx_b, acc_s, rows, sems, extra)

    plsc.subcore_barrier()

    iota16 = lax.iota(jnp.int32, 16)
    zero16 = jnp.zeros((16,), jnp.int32)
    dbufs = (rows0, rows1)
    wsems = (wsem0, wsem1)
    for k in range(10):
        b = k % 2
        rcnt = 128 if k < 9 else _MZ - 9 * 128
        off = sid * _MZ + k * 128
        if k >= 2:
            pltpu.make_async_copy(
                xe_out.at[pl.ds(0, 128)], dbufs[b].at[pl.ds(0, 128)],
                wsems[b]).wait()
        pltpu.sync_copy(acc_s.at[pl.ds(off, rcnt)], dbufs[b].at[pl.ds(0, rcnt)])
        pltpu.sync_copy(counts_s.at[pl.ds(off, rcnt)], aux.at[pl.ds(0, rcnt)])

        def dbody(g, carry, _b=b):
            cnt = plsc.load_gather(aux, [g * 16 + iota16, zero16])
            rcp = 1.0 / jnp.maximum(cnt, 1.0)
            for i in range(16):
                r = g * 16 + i
                rc = rcp[i]
                for c in range(_H // 16):
                    sl = pl.ds(c * 16, 16)
                    dbufs[_b][r, sl] = dbufs[_b][r, sl] * rc
            return carry

        lax.fori_loop(0, rcnt // 16, dbody, 0)
        pltpu.async_copy(dbufs[b].at[pl.ds(0, rcnt)],
                         xe_out.at[pl.ds(cid * _MG + off, rcnt)], wsems[b])
    pltpu.make_async_copy(
        xe_out.at[pl.ds(0, 128)], rows0.at[pl.ds(0, 128)], wsem0).wait()
    pltpu.make_async_copy(
        xe_out.at[pl.ds(0, _MZ - 9 * 128)],
        rows1.at[pl.ds(0, _MZ - 9 * 128)], wsem1).wait()

    plsc.subcore_barrier()

    pltpu.sync_copy(z64, rows0)
    for k in range(5):
        rcnt = 128 if k < 4 else _NZ - 4 * 128
        off = sid * _NZ + k * 128
        pltpu.sync_copy(rows0.at[pl.ds(0, rcnt)], acc_s.at[pl.ds(off, rcnt)])
    plsc.subcore_barrier()

    for ci in range(_NCH):
        base = sid * _RPT + ci * _CROWS
        pltpu.sync_copy(edg_g.at[pl.ds(cid * _IDX_ROWS + base, _CROWS)], idx_a)
        pltpu.sync_copy(vtx_s.at[pl.ds(base, _CROWS)], idx_b)
        _stream_pairs(xe_out, idx_a, idx_b, acc_s, rows, sems)

    plsc.subcore_barrier()
    pltpu.sync_copy(
        acc_s.at[pl.ds(sid * _NZ, _NZ)],
        xv_out.at[pl.ds(cid * _NG + sid * _NZ, _NZ)])


_BN4 = 1000


def _k4_body(x_ref, lo_ref, hi_ref, w_ref, e_ref, o_ref):
    xn = e_ref[0, 0] * x_ref[:, :] + jnp.concatenate(
        [lo_ref[:, :], hi_ref[:, :]], axis=1)
    o_ref[:, :] = lax.dot_general(
        xn, w_ref[:, :], (((1,), (1,)), ((), ())),
        preferred_element_type=jnp.float32)


def _k4(X, xv_lo, xv_hi, W, epsp1):
    nb = _N // _BN4
    return pl.pallas_call(
        _k4_body,
        grid=(nb,),
        in_specs=[
            pl.BlockSpec((_BN4, _D), lambda i: (i, 0)),
            pl.BlockSpec((_BN4, _H), lambda i: (i, 0)),
            pl.BlockSpec((_BN4, _H), lambda i: (i, 0)),
            pl.BlockSpec((_D, _D), lambda i: (0, 0)),
            pl.BlockSpec(memory_space=pltpu.SMEM),
        ],
        out_specs=pl.BlockSpec((_BN4, _D), lambda i: (i, 0)),
        out_shape=jax.ShapeDtypeStruct((_N, _D), jnp.float32),
    )(X, xv_lo, xv_hi, W, epsp1)


def kernel(X, vertex, edges, W, eps):
    pad = _P_PAD - _P
    ar = jnp.arange(pad, dtype=jnp.int32)
    vtx_g = jnp.concatenate([vertex, (ar * 7919) % _N])
    vtx_g2 = jnp.concatenate([vtx_g, vtx_g + _N]).reshape(2 * _IDX_ROWS, _LANES)
    edg_s = jnp.concatenate([edges, _M + (ar % (_MG - _M))]).reshape(
        _IDX_ROWS, _LANES)
    edg_g = jnp.concatenate([edges, (ar * 7919) % _M])
    edg_g2 = jnp.concatenate([edg_g, edg_g + _MG]).reshape(
        2 * _IDX_ROWS, _LANES)
    vtx_s = jnp.concatenate([vertex, _N + (ar % (_NG - _N))]).reshape(
        _IDX_ROWS, _LANES)

    xcat = jnp.concatenate([X[:, :_H], X[:, _H:]], axis=0)
    ones_h = jnp.ones((_LANES, _CW), jnp.float32)
    z64 = jnp.zeros((_LANES, _H), jnp.float32)
    zc = jnp.zeros((_LANES, _CW), jnp.float32)

    _, xv_cat = _kmain(xcat, vtx_g2, edg_s, edg_g2, vtx_s, ones_h, z64, zc)
    xv_lo = xv_cat[:_N]
    xv_hi = xv_cat[_NG:_NG + _N]
    epsp1 = (1.0 + eps).reshape(1, 1)
    return _k4(X, xv_lo, xv_hi, W, epsp1)
